# baseline (speedup 1.0000x reference)
import math

import jax
import jax.numpy as jnp
from jax.experimental import pallas as pl
from jax.experimental.pallas import tpu as pltpu

TOP_N = 64
_INT_MIN = -2147483648


def _f32_sort_key(x):
    u = jax.lax.bitcast_convert_type(x, jnp.int32)
    return jnp.where(u >= 0, u, u ^ jnp.int32(2147483647))


def _proj_body(xq_ref, peq_ref, xkv_ref, pekv_ref, wq_ref, wk_ref, q_ref, k_ref):
    pw = jnp.float32(0.8)
    xs = jnp.abs(1.0 - pw)
    rq = xq_ref[...] * xs + peq_ref[...] * pw
    rk = xkv_ref[...] * xs + pekv_ref[...] * pw
    q_ref[...] = jax.lax.dot_general(
        rq, wq_ref[...], (((1,), (1,)), ((), ())),
        precision=jax.lax.Precision.DEFAULT,
        preferred_element_type=jnp.float32) + rq
    k_ref[...] = jax.lax.dot_general(
        rk, wk_ref[...], (((1,), (1,)), ((), ())),
        precision=jax.lax.Precision.DEFAULT,
        preferred_element_type=jnp.float32) + rk


def _project(x_q, pe_q, x_kv, pe_kv, Wq, Wk):
    n, d = x_q.shape
    blk = min(512, n)
    nblk = n // blk
    full = pl.BlockSpec((d, d), lambda i: (0, 0))
    row = pl.BlockSpec((blk, d), lambda i: (i, 0))
    return pl.pallas_call(
        _proj_body,
        grid=(nblk,),
        in_specs=[row, row, row, row, full, full],
        out_specs=[row, row],
        out_shape=[jax.ShapeDtypeStruct((n, d), jnp.float32),
                   jax.ShapeDtypeStruct((n, d), jnp.float32)],
    )(x_q, pe_q, x_kv, pe_kv, Wq, Wk)


def _make_main(nq, nkv, d, blk_r, blk_c):
    nj = nkv // blk_c
    ni = nq // blk_r
    inv = 1.0 / math.sqrt(float(d))

    def body(peq_ref, q_ref, pekv_ref, k_ref, out_ref, strip_ref, thr_ref):
        j = pl.program_id(1)

        @pl.when(j < nj)
        def _pe_phase():
            s = jax.lax.dot_general(
                peq_ref[...], pekv_ref[...], (((1,), (1,)), ((), ())),
                precision=jax.lax.Precision.DEFAULT,
                preferred_element_type=jnp.float32) * inv
            strip_ref[j] = _f32_sort_key(s)

        @pl.when(j == nj)
        def _thresh_phase():
            def bit_body(b, prefix):
                one = jax.lax.shift_left(jnp.int32(1), 31 - b)
                cand = prefix | one
                tc = cand ^ jnp.int32(_INT_MIN)

                def chunk(jc, acc):
                    return acc + (strip_ref[jc] >= tc).astype(jnp.int32)

                acc = jax.lax.fori_loop(
                    0, nj, chunk, jnp.zeros((blk_r, blk_c), jnp.int32))
                cnt = jnp.sum(acc, axis=1, keepdims=True)
                return jnp.where(cnt >= TOP_N, cand, prefix)

            prefix = jax.lax.fori_loop(
                0, 32, bit_body, jnp.zeros((blk_r, 1), jnp.int32))
            thr_ref[...] = prefix ^ jnp.int32(_INT_MIN)

        @pl.when(j > nj)
        def _att_phase():
            jj = j - (nj + 1)
            att = jax.lax.dot_general(
                q_ref[...], k_ref[...], (((1,), (1,)), ((), ())),
                precision=jax.lax.Precision.DEFAULT,
                preferred_element_type=jnp.float32) * inv
            keep = strip_ref[jj] >= thr_ref[...]
            out_ref[...] = jnp.where(keep, att, 0.0)

    row = pl.BlockSpec((blk_r, d), lambda i, j: (i, 0))
    kv_pe = pl.BlockSpec((blk_c, d), lambda i, j: (jnp.minimum(j, nj - 1), 0))
    kv_k = pl.BlockSpec(
        (blk_c, d), lambda i, j: (jnp.clip(j - (nj + 1), 0, nj - 1), 0))
    out = pl.BlockSpec(
        (blk_r, blk_c), lambda i, j: (i, jnp.clip(j - (nj + 1), 0, nj - 1)))

    return pl.pallas_call(
        body,
        grid=(ni, 2 * nj + 1),
        in_specs=[row, row, kv_pe, kv_k],
        out_specs=out,
        out_shape=jax.ShapeDtypeStruct((nq, nkv), jnp.float32),
        scratch_shapes=[pltpu.VMEM((nj, blk_r, blk_c), jnp.int32),
                        pltpu.VMEM((blk_r, 1), jnp.int32)],
        compiler_params=pltpu.CompilerParams(
            dimension_semantics=("arbitrary", "arbitrary")),
    )


def kernel(x_q, x_kv, pe_q, pe_kv, Wq, Wk, Wv, bv):
    nq, d = x_q.shape
    nkv = x_kv.shape[0]
    query, key = _project(x_q, pe_q, x_kv, pe_kv, Wq, Wk)
    blk_r = min(512, nq)
    blk_c = min(1024, nkv)
    return _make_main(nq, nkv, d, blk_r, blk_c)(pe_q, query, pe_kv, key)


# speedup vs baseline: 1.9836x; 1.9836x over previous
import math

import jax
import jax.numpy as jnp
from jax.experimental import pallas as pl
from jax.experimental.pallas import tpu as pltpu

TOP_N = 64
_INT_MIN = -2147483648


def _f32_sort_key(x):
    u = jax.lax.bitcast_convert_type(x, jnp.int32)
    return jnp.where(u >= 0, u, u ^ jnp.int32(2147483647))


def _proj_body(xq_ref, peq_ref, xkv_ref, pekv_ref, wq_ref, wk_ref, q_ref, k_ref):
    pw = jnp.float32(0.8)
    xs = jnp.abs(1.0 - pw)
    rq = xq_ref[...] * xs + peq_ref[...] * pw
    rk = xkv_ref[...] * xs + pekv_ref[...] * pw
    q_ref[...] = jax.lax.dot_general(
        rq, wq_ref[...], (((1,), (1,)), ((), ())),
        precision=jax.lax.Precision.DEFAULT,
        preferred_element_type=jnp.float32) + rq
    k_ref[...] = jax.lax.dot_general(
        rk, wk_ref[...], (((1,), (1,)), ((), ())),
        precision=jax.lax.Precision.DEFAULT,
        preferred_element_type=jnp.float32) + rk


def _project(x_q, pe_q, x_kv, pe_kv, Wq, Wk):
    n, d = x_q.shape
    blk = min(512, n)
    nblk = n // blk
    full = pl.BlockSpec((d, d), lambda i: (0, 0))
    row = pl.BlockSpec((blk, d), lambda i: (i, 0))
    return pl.pallas_call(
        _proj_body,
        grid=(nblk,),
        in_specs=[row, row, row, row, full, full],
        out_specs=[row, row],
        out_shape=[jax.ShapeDtypeStruct((n, d), jnp.float32),
                   jax.ShapeDtypeStruct((n, d), jnp.float32)],
    )(x_q, pe_q, x_kv, pe_kv, Wq, Wk)


def _make_main(nq, nkv, d, blk_r, blk_c):
    nj = nkv // blk_c
    ni = nq // blk_r
    inv = 1.0 / math.sqrt(float(d))

    def body(peq_ref, q_ref, pekv_ref, k_ref, out_ref, strip_ref, thr_ref):
        j = pl.program_id(1)

        @pl.when(j < nj)
        def _pe_phase():
            s = jax.lax.dot_general(
                peq_ref[...], pekv_ref[...], (((1,), (1,)), ((), ())),
                precision=jax.lax.Precision.DEFAULT,
                preferred_element_type=jnp.float32) * inv
            strip_ref[j] = _f32_sort_key(s)

        @pl.when(j == nj)
        def _thresh_phase():
            def bit_body(b, prefix):
                one = jax.lax.shift_left(jnp.int32(1), 31 - b)
                cand = prefix | one
                tc = cand ^ jnp.int32(_INT_MIN)

                def chunk(jc, acc):
                    return acc + jnp.sum((strip_ref[jc] >= tc).astype(
                        jnp.int32), axis=1, keepdims=True)

                cnt = jax.lax.fori_loop(
                    0, nj, chunk, jnp.zeros((blk_r, 1), jnp.int32))
                return jnp.where(cnt >= TOP_N, cand, prefix)

            prefix = jax.lax.fori_loop(
                0, 32, bit_body, jnp.zeros((blk_r, 1), jnp.int32))
            thr_ref[...] = prefix ^ jnp.int32(_INT_MIN)

        @pl.when(j > nj)
        def _att_phase():
            jj = j - (nj + 1)
            att = jax.lax.dot_general(
                q_ref[...], k_ref[...], (((1,), (1,)), ((), ())),
                precision=jax.lax.Precision.DEFAULT,
                preferred_element_type=jnp.float32) * inv
            keep = strip_ref[jj] >= thr_ref[...]
            out_ref[...] = jnp.where(keep, att, 0.0)

    row = pl.BlockSpec((blk_r, d), lambda i, j: (i, 0))
    kv_pe = pl.BlockSpec((blk_c, d), lambda i, j: (jnp.minimum(j, nj - 1), 0))
    kv_k = pl.BlockSpec(
        (blk_c, d), lambda i, j: (jnp.clip(j - (nj + 1), 0, nj - 1), 0))
    out = pl.BlockSpec(
        (blk_r, blk_c), lambda i, j: (i, jnp.clip(j - (nj + 1), 0, nj - 1)))

    return pl.pallas_call(
        body,
        grid=(ni, 2 * nj + 1),
        in_specs=[row, row, kv_pe, kv_k],
        out_specs=out,
        out_shape=jax.ShapeDtypeStruct((nq, nkv), jnp.float32),
        scratch_shapes=[pltpu.VMEM((nj, blk_r, blk_c), jnp.int32),
                        pltpu.VMEM((blk_r, 1), jnp.int32)],
        compiler_params=pltpu.CompilerParams(
            dimension_semantics=("arbitrary", "arbitrary")),
    )


def kernel(x_q, x_kv, pe_q, pe_kv, Wq, Wk, Wv, bv):
    nq, d = x_q.shape
    nkv = x_kv.shape[0]
    query, key = _project(x_q, pe_q, x_kv, pe_kv, Wq, Wk)
    blk_r = min(512, nq)
    blk_c = min(2048, nkv)
    return _make_main(nq, nkv, d, blk_r, blk_c)(pe_q, query, pe_kv, key)


# speedup vs baseline: 2.2793x; 1.1491x over previous
import math

import jax
import jax.numpy as jnp
from jax.experimental import pallas as pl
from jax.experimental.pallas import tpu as pltpu

TOP_N = 64
_INT_MIN = -2147483648


def _f32_sort_key(x):
    u = jax.lax.bitcast_convert_type(x, jnp.int32)
    return jnp.where(u >= 0, u, u ^ jnp.int32(2147483647))


def _proj_body(xq_ref, peq_ref, xkv_ref, pekv_ref, wq_ref, wk_ref, q_ref, k_ref):
    pw = jnp.float32(0.8)
    xs = jnp.abs(1.0 - pw)
    rq = xq_ref[...] * xs + peq_ref[...] * pw
    rk = xkv_ref[...] * xs + pekv_ref[...] * pw
    q_ref[...] = jax.lax.dot_general(
        rq, wq_ref[...], (((1,), (1,)), ((), ())),
        precision=jax.lax.Precision.DEFAULT,
        preferred_element_type=jnp.float32) + rq
    k_ref[...] = jax.lax.dot_general(
        rk, wk_ref[...], (((1,), (1,)), ((), ())),
        precision=jax.lax.Precision.DEFAULT,
        preferred_element_type=jnp.float32) + rk


def _project(x_q, pe_q, x_kv, pe_kv, Wq, Wk):
    n, d = x_q.shape
    blk = min(512, n)
    nblk = n // blk
    full = pl.BlockSpec((d, d), lambda i: (0, 0))
    row = pl.BlockSpec((blk, d), lambda i: (i, 0))
    return pl.pallas_call(
        _proj_body,
        grid=(nblk,),
        in_specs=[row, row, row, row, full, full],
        out_specs=[row, row],
        out_shape=[jax.ShapeDtypeStruct((n, d), jnp.float32),
                   jax.ShapeDtypeStruct((n, d), jnp.float32)],
    )(x_q, pe_q, x_kv, pe_kv, Wq, Wk)


def _make_main(nq, nkv, d, blk_r, blk_c):
    nj = nkv // blk_c
    ni = nq // blk_r
    inv = 1.0 / math.sqrt(float(d))

    def body(peq_ref, q_ref, pekv_ref, k_ref, out_ref, strip_ref, thr_ref):
        j = pl.program_id(1)

        @pl.when(j < nj)
        def _pe_phase():
            s = jax.lax.dot_general(
                peq_ref[...], pekv_ref[...], (((1,), (1,)), ((), ())),
                precision=jax.lax.Precision.DEFAULT,
                preferred_element_type=jnp.float32) * inv
            strip_ref[j] = _f32_sort_key(s)

        @pl.when(j == nj)
        def _thresh_phase():
            def bit_body(b, prefix):
                one = jax.lax.shift_left(jnp.int32(1), 31 - b)
                cand = prefix | one
                tc = cand ^ jnp.int32(_INT_MIN)

                cnt = jnp.zeros((blk_r, 1), jnp.int32)
                for jc in range(nj):
                    cnt = cnt + jnp.sum((strip_ref[jc] >= tc).astype(
                        jnp.int32), axis=1, keepdims=True)
                return jnp.where(cnt >= TOP_N, cand, prefix)

            prefix = jax.lax.fori_loop(
                0, 32, bit_body, jnp.zeros((blk_r, 1), jnp.int32))
            thr_ref[...] = prefix ^ jnp.int32(_INT_MIN)

        @pl.when(j > nj)
        def _att_phase():
            jj = j - (nj + 1)
            att = jax.lax.dot_general(
                q_ref[...], k_ref[...], (((1,), (1,)), ((), ())),
                precision=jax.lax.Precision.DEFAULT,
                preferred_element_type=jnp.float32) * inv
            keep = strip_ref[jj] >= thr_ref[...]
            out_ref[...] = jnp.where(keep, att, 0.0)

    row = pl.BlockSpec((blk_r, d), lambda i, j: (i, 0))
    kv_pe = pl.BlockSpec((blk_c, d), lambda i, j: (jnp.minimum(j, nj - 1), 0))
    kv_k = pl.BlockSpec(
        (blk_c, d), lambda i, j: (jnp.clip(j - (nj + 1), 0, nj - 1), 0))
    out = pl.BlockSpec(
        (blk_r, blk_c), lambda i, j: (i, jnp.clip(j - (nj + 1), 0, nj - 1)))

    return pl.pallas_call(
        body,
        grid=(ni, 2 * nj + 1),
        in_specs=[row, row, kv_pe, kv_k],
        out_specs=out,
        out_shape=jax.ShapeDtypeStruct((nq, nkv), jnp.float32),
        scratch_shapes=[pltpu.VMEM((nj, blk_r, blk_c), jnp.int32),
                        pltpu.VMEM((blk_r, 1), jnp.int32)],
        compiler_params=pltpu.CompilerParams(
            dimension_semantics=("arbitrary", "arbitrary")),
    )


def kernel(x_q, x_kv, pe_q, pe_kv, Wq, Wk, Wv, bv):
    nq, d = x_q.shape
    nkv = x_kv.shape[0]
    query, key = _project(x_q, pe_q, x_kv, pe_kv, Wq, Wk)
    blk_r = min(512, nq)
    blk_c = min(2048, nkv)
    return _make_main(nq, nkv, d, blk_r, blk_c)(pe_q, query, pe_kv, key)
